# baseline (speedup 1.0000x reference)
import jax
import jax.numpy as jnp
from jax.experimental import pallas as pl
from jax.experimental.pallas import tpu as pltpu

_EMB = 32
_NUM = 1024
_B = 8
_S = 8 * 32 * 32
_SB = 8192
_NBLK = _S // _SB


def _vq_kernel(x_ref, e_ref, q_ref, ind_ref, acc_ref):
    e = e_ref[...]

    e2 = jnp.sum(e * e, axis=0)[:, None]
    iota = jax.lax.broadcasted_iota(jnp.int32, (1, _NUM), 1).astype(jnp.float32)
    g_aug = jnp.concatenate([e, iota], axis=0)

    _HF = _SB // 2
    total = jnp.zeros((1, 1), jnp.float32)
    for h in range(2):
        sl = pl.ds(h * _HF, _HF)
        x = x_ref[0, :, sl]
        prod = jax.lax.dot_general(-2.0 * e, x, (((0,), (0,)), ((), ())),
                                   preferred_element_type=jnp.float32)
        scores = prod + e2

        m = jnp.min(scores, axis=0)[None, :]
        mask = (scores <= m).astype(jnp.float32)

        out = jax.lax.dot_general(g_aug, mask, (((1,), (0,)), ((), ())),
                                  preferred_element_type=jnp.float32)
        q = out[:_EMB]
        ind_ref[0, 0, 0, sl] = out[_EMB].astype(jnp.int32)

        q_ref[0, :, sl] = x + (q - x)
        total = total + jnp.sum((q - x) ** 2).reshape(1, 1)

    b = pl.program_id(0)

    @pl.when(b == 0)
    def _():
        acc_ref[...] = jnp.zeros_like(acc_ref)

    acc_ref[...] += total

    @pl.when(b == _B - 1)
    def _():
        acc_ref[...] = acc_ref[...] * (1.0 / (_B * _S * _EMB))


def kernel(input, embedding):
    x = input.reshape(_B, _EMB, _S)

    quant, ind, acc = pl.pallas_call(
        _vq_kernel,
        grid=(_B, _NBLK),
        in_specs=[
            pl.BlockSpec((1, _EMB, _SB), lambda b, s: (b, 0, s)),
            pl.BlockSpec((_EMB, _NUM), lambda b, s: (0, 0)),
        ],
        out_specs=[
            pl.BlockSpec((1, _EMB, _SB), lambda b, s: (b, 0, s)),
            pl.BlockSpec((1, 1, 1, _SB), lambda b, s: (b, s, 0, 0)),
            pl.BlockSpec((1, 1), lambda b, s: (0, 0)),
        ],
        out_shape=[
            jax.ShapeDtypeStruct((_B, _EMB, _S), jnp.float32),
            jax.ShapeDtypeStruct((_B, _NBLK, 1, _SB), jnp.int32),
            jax.ShapeDtypeStruct((1, 1), jnp.float32),
        ],
        compiler_params=pltpu.CompilerParams(
            dimension_semantics=("arbitrary", "arbitrary"),
            vmem_limit_bytes=100 * 1024 * 1024,
        ),
    )(x, embedding)

    quantize = quant.reshape(input.shape)
    diff = acc[0, 0]
    embedding_ind = ind.reshape(_B, 8, 32, 32)
    return quantize, diff, embedding_ind


# speedup vs baseline: 1.0024x; 1.0024x over previous
import jax
import jax.numpy as jnp
from jax.experimental import pallas as pl
from jax.experimental.pallas import tpu as pltpu

_EMB = 32
_NUM = 1024
_B = 8
_S = 8 * 32 * 32
_SB = 8192
_NBLK = _S // _SB


def _vq_kernel(x_ref, e_ref, q_ref, ind_ref, acc_ref):
    e = e_ref[...]

    e2 = jnp.sum(e * e, axis=0)[:, None]
    iota = jax.lax.broadcasted_iota(jnp.int32, (1, _NUM), 1).astype(jnp.float32)
    g_aug = jnp.concatenate([e, iota], axis=0)

    _HF = _SB // 4
    total = jnp.zeros((1, 1), jnp.float32)
    for h in range(4):
        sl = pl.ds(h * _HF, _HF)
        x = x_ref[0, :, sl]
        prod = jax.lax.dot_general(-2.0 * e, x, (((0,), (0,)), ((), ())),
                                   preferred_element_type=jnp.float32)
        scores = prod + e2

        m = jnp.min(scores, axis=0)[None, :]
        mask = (scores <= m).astype(jnp.float32)

        out = jax.lax.dot_general(g_aug, mask, (((1,), (0,)), ((), ())),
                                  preferred_element_type=jnp.float32)
        q = out[:_EMB]
        ind_ref[0, 0, 0, sl] = out[_EMB].astype(jnp.int32)

        q_ref[0, :, sl] = x + (q - x)
        total = total + jnp.sum((q - x) ** 2).reshape(1, 1)

    b = pl.program_id(0)

    @pl.when(b == 0)
    def _():
        acc_ref[...] = jnp.zeros_like(acc_ref)

    acc_ref[...] += total

    @pl.when(b == _B - 1)
    def _():
        acc_ref[...] = acc_ref[...] * (1.0 / (_B * _S * _EMB))


def kernel(input, embedding):
    x = input.reshape(_B, _EMB, _S)

    quant, ind, acc = pl.pallas_call(
        _vq_kernel,
        grid=(_B, _NBLK),
        in_specs=[
            pl.BlockSpec((1, _EMB, _SB), lambda b, s: (b, 0, s)),
            pl.BlockSpec((_EMB, _NUM), lambda b, s: (0, 0)),
        ],
        out_specs=[
            pl.BlockSpec((1, _EMB, _SB), lambda b, s: (b, 0, s)),
            pl.BlockSpec((1, 1, 1, _SB), lambda b, s: (b, s, 0, 0)),
            pl.BlockSpec((1, 1), lambda b, s: (0, 0)),
        ],
        out_shape=[
            jax.ShapeDtypeStruct((_B, _EMB, _S), jnp.float32),
            jax.ShapeDtypeStruct((_B, _NBLK, 1, _SB), jnp.int32),
            jax.ShapeDtypeStruct((1, 1), jnp.float32),
        ],
        compiler_params=pltpu.CompilerParams(
            dimension_semantics=("arbitrary", "arbitrary"),
            vmem_limit_bytes=100 * 1024 * 1024,
        ),
    )(x, embedding)

    quantize = quant.reshape(input.shape)
    diff = acc[0, 0]
    embedding_ind = ind.reshape(_B, 8, 32, 32)
    return quantize, diff, embedding_ind
